# baseline (speedup 1.0000x reference)
import functools

import jax
import jax.numpy as jnp
from jax import lax
from jax.experimental import pallas as pl
from jax.experimental.pallas import tpu as pltpu
from jax.experimental.pallas import tpu_sc as plsc

N = 10000
E = 320000
EMBED = 128
HIDDEN = 128
NREL = 8
NCLS = 16

NC = 2
NS = 16
NW = NC * NS
EPW = E // NW
EPT = E // NS

BIN_R = 640
BIN_C = 128

C2 = 2000
G = 80
Q = 5

_mesh = plsc.VectorSubcoreMesh(
    core_axis_name="c", subcore_axis_name="s", num_cores=NC, num_subcores=NS)
_sc_params = pltpu.CompilerParams(needs_layout_passes=False)
_sc_params_untiled = pltpu.CompilerParams(needs_layout_passes=False,
                                          use_tc_tiling_on_sc=False)

_f32 = jnp.float32
_i32 = jnp.int32


def _iota16():
    return lax.broadcasted_iota(_i32, (16,), 0)


def _zeros16():
    return jnp.zeros((16,), _f32)


@functools.partial(
    pl.kernel,
    out_type=jax.ShapeDtypeStruct((E,), _f32),
    mesh=_mesh,
    compiler_params=_sc_params,
    scratch_types=[
        pltpu.VMEM((BIN_R, BIN_C), _f32),
        pltpu.VMEM((BIN_R // 128, 128), _i32),
        pltpu.VMEM((C2,), _i32),
        pltpu.VMEM((C2,), _i32),
        pltpu.VMEM((BIN_R // NS, BIN_C), _f32),
        pltpu.VMEM((C2,), _f32),
        pltpu.VMEM_SHARED((BIN_R, BIN_C), _f32),
    ],
)
def _count_w(dst_hbm, rel_hbm, w_hbm, cntv, idxv, db, rb, cinvb, wb, shared):
    c = lax.axis_index("c")
    s = lax.axis_index("s")
    rps = BIN_R // NS

    def zrow(i, _):
        for m in range(BIN_C // 16):
            cntv[i, pl.ds(m * 16, 16)] = _zeros16()
        return 0
    lax.fori_loop(0, BIN_R, zrow, 0)

    pltpu.sync_copy(cntv.at[pl.ds(0, rps)], shared.at[pl.ds(s * rps, rps)])

    def count_chunk(ch, _):
        base = s * EPT + ch * C2
        pltpu.sync_copy(dst_hbm.at[pl.ds(base, C2)], db)
        pltpu.sync_copy(rel_hbm.at[pl.ds(base, C2)], rb)

        def cgrp(j, _):
            d16 = db[pl.ds(j * 16, 16)]
            r16 = rb[pl.ds(j * 16, 16)]
            b16 = d16 * NREL + r16
            ro = b16 >> 7
            co = b16 & (BIN_C - 1)
            ones = jnp.ones((16,), _f32)
            for k in range(16):
                plsc.addupdate_scatter(cntv, [ro, co], ones,
                                       mask=_iota16() == k)
            return 0
        lax.fori_loop(0, C2 // 16, cgrp, 0)
        return 0
    lax.fori_loop(0, EPT // C2, count_chunk, 0)

    plsc.subcore_barrier()

    def irow(j, _):
        for m in range(128 // 16):
            idxv[j, pl.ds(m * 16, 16)] = _iota16() + (j * 128 + m * 16)
        return 0
    lax.fori_loop(0, BIN_R // 128, irow, 0)

    def radd(j, _):
        pltpu.sync_copy(cntv.at[pl.ds(j * 128, 128)], shared.at[idxv.at[j]],
                        add=True)
        return 0
    lax.fori_loop(0, BIN_R // 128, radd, 0)

    plsc.subcore_barrier()

    pltpu.sync_copy(shared.at[pl.ds(s * rps, rps)], cinvb)

    def invrow(i, _):
        for m in range(BIN_C // 16):
            v = cinvb[i, pl.ds(m * 16, 16)]
            cinvb[i, pl.ds(m * 16, 16)] = 1.0 / jnp.maximum(v, 1.0)
        return 0
    lax.fori_loop(0, rps, invrow, 0)
    pltpu.sync_copy(cinvb, shared.at[pl.ds(s * rps, rps)])

    plsc.subcore_barrier()

    pltpu.sync_copy(shared, cntv)

    wid = s * NC + c

    def wchunk(ch, _):
        base = wid * EPW + ch * C2
        pltpu.sync_copy(dst_hbm.at[pl.ds(base, C2)], db)
        pltpu.sync_copy(rel_hbm.at[pl.ds(base, C2)], rb)

        def wgrp(j, _):
            d16 = db[pl.ds(j * 16, 16)]
            r16 = rb[pl.ds(j * 16, 16)]
            b16 = d16 * NREL + r16
            ro = b16 >> 7
            co = b16 & (BIN_C - 1)
            wb[pl.ds(j * 16, 16)] = plsc.load_gather(cntv, [ro, co])
            return 0
        lax.fori_loop(0, C2 // 16, wgrp, 0)
        pltpu.sync_copy(wb, w_hbm.at[pl.ds(base, C2)])
        return 0
    lax.fori_loop(0, EPW // C2, wchunk, 0)


def _make_agg(D):
    params = _sc_params if D == 128 else _sc_params_untiled
    zr = 80
    nzc = N // zr
    zpt = (nzc + NS - 1) // NS
    SB = Q * G
    nsp = EPW // SB

    @functools.partial(
        pl.kernel,
        out_type=jax.ShapeDtypeStruct((NC, N, D), _f32),
        mesh=_mesh,
        compiler_params=params,
        scratch_types=[
            pltpu.VMEM((SB,), _i32),
            pltpu.VMEM((SB,), _i32),
            pltpu.VMEM((Q, G), _i32),
            pltpu.VMEM((Q, G), _i32),
            pltpu.VMEM((SB,), _f32),
            pltpu.VMEM((G, D), _f32),
            pltpu.VMEM((G, D), _f32),
            pltpu.VMEM_SHARED((N, D), _f32),
            pltpu.SemaphoreType.DMA,
            pltpu.SemaphoreType.DMA,
            pltpu.SemaphoreType.DMA,
            pltpu.SemaphoreType.DMA,
            pltpu.SemaphoreType.DMA,
        ],
    )
    def agg(hr_hbm, w_hbm, src_hbm, dst_hbm, rel_hbm, part_hbm,
            srcb, relb, dstb, gidxb, wb, rows0, rows1, acc,
            semL, semG0, semG1, semS0, semS1):
        c = lax.axis_index("c")
        s = lax.axis_index("s")
        wid = s * NC + c
        bufs = [rows0, rows1]
        sems = [semG0, semG1]
        semS = [semS0, semS1]

        def zrow(i, _):
            for m in range(D // 16):
                rows0[i, pl.ds(m * 16, 16)] = _zeros16()
            return 0
        lax.fori_loop(0, zr, zrow, 0)

        for t in range(zpt):
            ci = s * zpt + t

            @pl.when(ci < nzc)
            def _():
                pltpu.sync_copy(rows0, acc.at[pl.ds(ci * zr, zr)])
        plsc.subcore_barrier()

        def chunk(sup, _):
            base = wid * EPW + sup * SB
            cps = [
                pltpu.async_copy(src_hbm.at[pl.ds(base, SB)], srcb, semL),
                pltpu.async_copy(rel_hbm.at[pl.ds(base, SB)], relb, semL),
                pltpu.async_copy(w_hbm.at[pl.ds(base, SB)], wb, semL),
            ] + [
                pltpu.async_copy(dst_hbm.at[pl.ds(base + q * G, G)],
                                 dstb.at[q], semL)
                for q in range(Q)
            ]
            for cp in cps:
                cp.wait()
            gcps = []
            for q in range(2):
                for m in range(G // 16):
                    o = q * G + m * 16
                    gidxb[q, pl.ds(m * 16, 16)] = (
                        srcb[pl.ds(o, 16)] * NREL + relb[pl.ds(o, 16)])
                gcps.append(pltpu.async_copy(hr_hbm.at[gidxb.at[q]],
                                             bufs[q % 2], sems[q % 2]))
            for q in range(2, Q):
                for m in range(G // 16):
                    o = q * G + m * 16
                    gidxb[q, pl.ds(m * 16, 16)] = (
                        srcb[pl.ds(o, 16)] * NREL + relb[pl.ds(o, 16)])
            pend_s = [None, None]
            for q in range(Q):
                b = q % 2
                gcps[q].wait()

                def sgrp(j, _):
                    w16 = wb[pl.ds(q * G + j * 16, 16)]
                    for k in range(16):
                        e = j * 16 + k
                        wsc = w16[k]
                        for m in range(D // 16):
                            bufs[b][e, pl.ds(m * 16, 16)] = (
                                bufs[b][e, pl.ds(m * 16, 16)] * wsc)
                    return 0
                lax.fori_loop(0, G // 16, sgrp, 0)

                pend_s[b] = pltpu.async_copy(bufs[b], acc.at[dstb.at[q]],
                                             semS[b], add=True)
                if q + 2 < Q:
                    nb = (q + 2) % 2
                    pend_s[nb].wait()
                    gcps.append(pltpu.async_copy(hr_hbm.at[gidxb.at[q + 2]],
                                                 bufs[nb], sems[nb]))
            for p in pend_s:
                p.wait()
            return 0
        lax.fori_loop(0, nsp, chunk, 0)

        plsc.subcore_barrier()
        for t in range(zpt):
            ci = s * zpt + t

            @pl.when(ci < nzc)
            def _():
                pltpu.sync_copy(acc.at[pl.ds(ci * zr, zr)],
                                part_hbm.at[c, pl.ds(ci * zr, zr)])

    return agg


_agg_full = _make_agg(128)
_agg_narrow = _make_agg(NCLS)


BN = 2000


def _prelu(v, a):
    return jnp.where(v >= 0, v, a * v)


def _combine_w(comp_ref, bases_ref):
    ws = []
    for r in range(NREL):
        W = comp_ref[r, 0] * bases_ref[0]
        for b in range(1, NREL):
            W = W + comp_ref[r, b] * bases_ref[b]
        ws.append(W)
    return ws


def _store_hr(hr_ref, r, hrr, slot):
    hr_ref[:, r, :] = hrr


def _mm1_body(num_ref, numm_ref, txt_ref, txtm_ref, x_ref,
              npw_ref, npb_ref, tpw_ref, tpb_ref, xpw_ref, xpb_ref, a_ref,
              comp_ref, bases_ref, root_ref, bias_ref, hr_ref,
              base_ref, *, slot):
    a = a_ref[...]
    nm = num_ref[...] * numm_ref[...]
    h0 = _prelu(nm * npw_ref[...] + npb_ref[...], a)
    t = jnp.dot(txt_ref[...] * txtm_ref[...], tpw_ref[...],
                preferred_element_type=_f32)
    t = _prelu(t + tpb_ref[...], a)
    xp = jnp.dot(x_ref[...], xpw_ref[...], preferred_element_type=_f32)
    h = h0 + t + xp + xpb_ref[...]
    ws = _combine_w(comp_ref, bases_ref)
    for r in range(NREL):
        hrr = jnp.dot(h, ws[r], preferred_element_type=_f32)
        _store_hr(hr_ref, r, hrr, slot)
    base_ref[...] = (jnp.dot(h, root_ref[...], preferred_element_type=_f32)
                     + bias_ref[...])


def _mmf_body(basep_ref, part_ref, act_ref, comp_ref, bases_ref, root_ref,
              bias_ref, hr_ref, base_ref, *, slot):
    p = part_ref[...]
    h = _prelu(basep_ref[...] + p[0] + p[1], act_ref[...])
    ws = _combine_w(comp_ref, bases_ref)
    for r in range(NREL):
        hrr = jnp.dot(h, ws[r], preferred_element_type=_f32)
        _store_hr(hr_ref, r, hrr, slot)
    base_ref[...] = (jnp.dot(h, root_ref[...], preferred_element_type=_f32)
                     + bias_ref[...])


def _hr_out(slot):
    d = NCLS if slot else HIDDEN
    return (pl.BlockSpec((BN, NREL, d), lambda i: (i, 0, 0)),
            jax.ShapeDtypeStruct((N, NREL, d), _f32))


def _mm1_call(num_x, num_mask, txt_x, txt_mask, x,
              npw, npb, tpw, tpb, xpw, xpb, a,
              comp, bases, root, bias, F, D, slot):
    grid = (N // BN,)
    blk = lambda shape: pl.BlockSpec(shape, lambda i: (i,) + (0,) * (len(shape) - 1))
    full = lambda shape: pl.BlockSpec(shape, lambda i: (0,) * len(shape))
    hr_spec, hr_shape = _hr_out(slot)
    return pl.pallas_call(
        functools.partial(_mm1_body, slot=slot),
        grid=grid,
        in_specs=[
            blk((BN, 1)), blk((BN, 1)), blk((BN, 384)), blk((BN, 1)),
            blk((BN, EMBED)),
            full((1, EMBED)), full((1, EMBED)), full((384, EMBED)),
            full((1, EMBED)), full((EMBED, HIDDEN)), full((1, HIDDEN)),
            full((1, EMBED)),
            pl.BlockSpec((NREL, NREL), lambda i: (0, 0),
                         memory_space=pltpu.SMEM),
            full((NREL, F, D)), full((F, D)), full((1, D)),
        ],
        out_specs=[hr_spec, blk((BN, D))],
        out_shape=[hr_shape, jax.ShapeDtypeStruct((N, D), _f32)],
    )(num_x, num_mask, txt_x, txt_mask, x, npw, npb, tpw, tpb, xpw, xpb, a,
      comp, bases, root, bias)


def _mmf_call(basep, part, act, comp, bases, root, bias, F, D, slot):
    grid = (N // BN,)
    blk = lambda shape: pl.BlockSpec(shape, lambda i: (i,) + (0,) * (len(shape) - 1))
    full = lambda shape: pl.BlockSpec(shape, lambda i: (0,) * len(shape))
    hr_spec, hr_shape = _hr_out(slot)
    return pl.pallas_call(
        functools.partial(_mmf_body, slot=slot),
        grid=grid,
        in_specs=[
            blk((BN, F)),
            pl.BlockSpec((NC, BN, 128), lambda i: (0, i, 0)),
            full((1, F)),
            pl.BlockSpec((NREL, NREL), lambda i: (0, 0),
                         memory_space=pltpu.SMEM),
            full((NREL, F, D)), full((F, D)), full((1, D)),
        ],
        out_specs=[hr_spec, blk((BN, D))],
        out_shape=[hr_shape, jax.ShapeDtypeStruct((N, D), _f32)],
    )(basep, part, act, comp, bases, root, bias)


def _final_body(base_ref, part_ref, out_ref):
    p = part_ref[...]
    v = base_ref[...] + p[0] + p[1]
    m = jnp.max(v, axis=1, keepdims=True)
    z = v - m
    lse = jnp.log(jnp.sum(jnp.exp(z), axis=1, keepdims=True))
    out_ref[...] = z - lse


def _final_call(base, part):
    return pl.pallas_call(
        _final_body,
        out_shape=jax.ShapeDtypeStruct((N, NCLS), _f32),
    )(base, part)


def kernel(x, num_x, num_mask, txt_x, txt_mask, edge_index, edge_type,
           num_proj_w, num_proj_b, txt_proj_w, txt_proj_b,
           node_proj_w, node_proj_b,
           input_act_a, act1_a, act2_a,
           comp1, bases1, root1, bias1,
           comp2, bases2, root2, bias2,
           comp3, bases3, root3, bias3):
    src = edge_index[0]
    dst = edge_index[1]
    et = edge_type

    w = _count_w(dst, et)

    hr1, b1 = _mm1_call(num_x, num_mask, txt_x, txt_mask.reshape(N, 1), x,
                        num_proj_w, num_proj_b.reshape(1, EMBED),
                        txt_proj_w, txt_proj_b.reshape(1, EMBED),
                        node_proj_w, node_proj_b.reshape(1, HIDDEN),
                        input_act_a.reshape(1, EMBED),
                        comp1, bases1, root1, bias1.reshape(1, HIDDEN),
                        EMBED, HIDDEN, False)
    p1 = _agg_full(hr1.reshape(N * NREL, HIDDEN), w, src, dst, et)

    hr2, b2 = _mmf_call(b1, p1, act1_a.reshape(1, HIDDEN), comp2, bases2,
                        root2, bias2.reshape(1, HIDDEN), HIDDEN, HIDDEN,
                        False)
    p2 = _agg_full(hr2.reshape(N * NREL, HIDDEN), w, src, dst, et)

    hr3, b3 = _mmf_call(b2, p2, act2_a.reshape(1, HIDDEN), comp3, bases3,
                        root3, bias3.reshape(1, NCLS), HIDDEN, NCLS, True)
    p3 = _agg_narrow(hr3.reshape(N * NREL, NCLS), w, src, dst, et)

    return _final_call(b3, p3)


# speedup vs baseline: 1.0539x; 1.0539x over previous
import functools

import jax
import jax.numpy as jnp
from jax import lax
from jax.experimental import pallas as pl
from jax.experimental.pallas import tpu as pltpu
from jax.experimental.pallas import tpu_sc as plsc

N = 10000
E = 320000
EMBED = 128
HIDDEN = 128
NREL = 8
NCLS = 16

NC = 2
NS = 16
NW = NC * NS
EPW = E // NW
EPT = E // NS

BIN_R = 640
BIN_C = 128

C2 = 2000
G = 80
Q = 5

_mesh = plsc.VectorSubcoreMesh(
    core_axis_name="c", subcore_axis_name="s", num_cores=NC, num_subcores=NS)
_sc_params = pltpu.CompilerParams(needs_layout_passes=False)
_sc_params_untiled = pltpu.CompilerParams(needs_layout_passes=False,
                                          use_tc_tiling_on_sc=False)

_f32 = jnp.float32
_i32 = jnp.int32


def _iota16():
    return lax.broadcasted_iota(_i32, (16,), 0)


def _zeros16():
    return jnp.zeros((16,), _f32)


@functools.partial(
    pl.kernel,
    out_type=jax.ShapeDtypeStruct((E,), _f32),
    mesh=_mesh,
    compiler_params=_sc_params,
    scratch_types=[
        pltpu.VMEM((BIN_R, BIN_C), _f32),
        pltpu.VMEM((BIN_R // 128, 128), _i32),
        pltpu.VMEM((C2,), _i32),
        pltpu.VMEM((C2,), _i32),
        pltpu.VMEM((BIN_R // NS, BIN_C), _f32),
        pltpu.VMEM((C2,), _f32),
        pltpu.VMEM_SHARED((BIN_R, BIN_C), _f32),
    ],
)
def _count_w(dst_hbm, rel_hbm, w_hbm, cntv, idxv, db, rb, cinvb, wb, shared):
    c = lax.axis_index("c")
    s = lax.axis_index("s")
    rps = BIN_R // NS

    def zrow(i, _):
        for m in range(BIN_C // 16):
            cntv[i, pl.ds(m * 16, 16)] = _zeros16()
        return 0
    lax.fori_loop(0, BIN_R, zrow, 0)

    pltpu.sync_copy(cntv.at[pl.ds(0, rps)], shared.at[pl.ds(s * rps, rps)])

    def count_chunk(ch, _):
        base = s * EPT + ch * C2
        pltpu.sync_copy(dst_hbm.at[pl.ds(base, C2)], db)
        pltpu.sync_copy(rel_hbm.at[pl.ds(base, C2)], rb)

        def cgrp(j, _):
            d16 = db[pl.ds(j * 16, 16)]
            r16 = rb[pl.ds(j * 16, 16)]
            b16 = d16 * NREL + r16
            ro = b16 >> 7
            co = b16 & (BIN_C - 1)
            ones = jnp.ones((16,), _f32)
            for k in range(16):
                plsc.addupdate_scatter(cntv, [ro, co], ones,
                                       mask=_iota16() == k)
            return 0
        lax.fori_loop(0, C2 // 16, cgrp, 0)
        return 0
    lax.fori_loop(0, EPT // C2, count_chunk, 0)

    plsc.subcore_barrier()

    def irow(j, _):
        for m in range(128 // 16):
            idxv[j, pl.ds(m * 16, 16)] = _iota16() + (j * 128 + m * 16)
        return 0
    lax.fori_loop(0, BIN_R // 128, irow, 0)

    def radd(j, _):
        pltpu.sync_copy(cntv.at[pl.ds(j * 128, 128)], shared.at[idxv.at[j]],
                        add=True)
        return 0
    lax.fori_loop(0, BIN_R // 128, radd, 0)

    plsc.subcore_barrier()

    pltpu.sync_copy(shared.at[pl.ds(s * rps, rps)], cinvb)

    def invrow(i, _):
        for m in range(BIN_C // 16):
            v = cinvb[i, pl.ds(m * 16, 16)]
            cinvb[i, pl.ds(m * 16, 16)] = 1.0 / jnp.maximum(v, 1.0)
        return 0
    lax.fori_loop(0, rps, invrow, 0)
    pltpu.sync_copy(cinvb, shared.at[pl.ds(s * rps, rps)])

    plsc.subcore_barrier()

    pltpu.sync_copy(shared, cntv)

    wid = s * NC + c

    def wchunk(ch, _):
        base = wid * EPW + ch * C2
        pltpu.sync_copy(dst_hbm.at[pl.ds(base, C2)], db)
        pltpu.sync_copy(rel_hbm.at[pl.ds(base, C2)], rb)

        def wgrp(j, _):
            d16 = db[pl.ds(j * 16, 16)]
            r16 = rb[pl.ds(j * 16, 16)]
            b16 = d16 * NREL + r16
            ro = b16 >> 7
            co = b16 & (BIN_C - 1)
            wb[pl.ds(j * 16, 16)] = plsc.load_gather(cntv, [ro, co])
            return 0
        lax.fori_loop(0, C2 // 16, wgrp, 0)
        pltpu.sync_copy(wb, w_hbm.at[pl.ds(base, C2)])
        return 0
    lax.fori_loop(0, EPW // C2, wchunk, 0)


def _make_agg(D):
    params = _sc_params if D == 128 else _sc_params_untiled
    zr = 80
    nzc = N // zr
    zpt = (nzc + NS - 1) // NS
    SB = Q * G
    nsp = EPW // SB

    @functools.partial(
        pl.kernel,
        out_type=jax.ShapeDtypeStruct((NC, N, D), _f32),
        mesh=_mesh,
        compiler_params=params,
        scratch_types=[
            pltpu.VMEM((SB,), _i32),
            pltpu.VMEM((SB,), _i32),
            pltpu.VMEM((Q, G), _i32),
            pltpu.VMEM((Q, G), _i32),
            pltpu.VMEM((SB,), _f32),
            pltpu.VMEM((G, D), _f32),
            pltpu.VMEM((G, D), _f32),
            pltpu.VMEM((G, D), _f32),
            pltpu.VMEM_SHARED((N, D), _f32),
            pltpu.SemaphoreType.DMA,
            pltpu.SemaphoreType.DMA,
            pltpu.SemaphoreType.DMA,
            pltpu.SemaphoreType.DMA,
            pltpu.SemaphoreType.DMA,
            pltpu.SemaphoreType.DMA,
            pltpu.SemaphoreType.DMA,
        ],
    )
    def agg(hr_hbm, w_hbm, src_hbm, dst_hbm, rel_hbm, part_hbm,
            srcb, relb, dstb, gidxb, wb, rows0, rows1, rows2, acc,
            semL, semG0, semG1, semG2, semS0, semS1, semS2):
        c = lax.axis_index("c")
        s = lax.axis_index("s")
        wid = s * NC + c
        bufs = [rows0, rows1, rows2]
        sems = [semG0, semG1, semG2]
        semS = [semS0, semS1, semS2]

        def zrow(i, _):
            for m in range(D // 16):
                rows0[i, pl.ds(m * 16, 16)] = _zeros16()
            return 0
        lax.fori_loop(0, zr, zrow, 0)

        for t in range(zpt):
            ci = s * zpt + t

            @pl.when(ci < nzc)
            def _():
                pltpu.sync_copy(rows0, acc.at[pl.ds(ci * zr, zr)])
        plsc.subcore_barrier()

        def chunk(sup, _):
            base = wid * EPW + sup * SB
            cps = [
                pltpu.async_copy(src_hbm.at[pl.ds(base, SB)], srcb, semL),
                pltpu.async_copy(rel_hbm.at[pl.ds(base, SB)], relb, semL),
                pltpu.async_copy(w_hbm.at[pl.ds(base, SB)], wb, semL),
            ] + [
                pltpu.async_copy(dst_hbm.at[pl.ds(base + q * G, G)],
                                 dstb.at[q], semL)
                for q in range(Q)
            ]
            for cp in cps:
                cp.wait()
            gcps = []
            for q in range(Q):
                for m in range(G // 16):
                    o = q * G + m * 16
                    gidxb[q, pl.ds(m * 16, 16)] = (
                        srcb[pl.ds(o, 16)] * NREL + relb[pl.ds(o, 16)])
                if q < 2:
                    gcps.append(pltpu.async_copy(hr_hbm.at[gidxb.at[q]],
                                                 bufs[q % 3], sems[q % 3]))
            pend_s = [None, None, None]
            for q in range(Q):
                b = q % 3
                if q + 2 < Q:
                    nb = (q + 2) % 3
                    if pend_s[nb] is not None:
                        pend_s[nb].wait()
                        pend_s[nb] = None
                    gcps.append(pltpu.async_copy(hr_hbm.at[gidxb.at[q + 2]],
                                                 bufs[nb], sems[nb]))
                gcps[q].wait()

                def sgrp(j, _):
                    w16 = wb[pl.ds(q * G + j * 16, 16)]
                    for k in range(16):
                        e = j * 16 + k
                        wsc = w16[k]
                        for m in range(D // 16):
                            bufs[b][e, pl.ds(m * 16, 16)] = (
                                bufs[b][e, pl.ds(m * 16, 16)] * wsc)
                    return 0
                lax.fori_loop(0, G // 16, sgrp, 0)

                pend_s[b] = pltpu.async_copy(bufs[b], acc.at[dstb.at[q]],
                                             semS[b], add=True)
            for p in pend_s:
                if p is not None:
                    p.wait()
            return 0
        lax.fori_loop(0, nsp, chunk, 0)

        plsc.subcore_barrier()
        for t in range(zpt):
            ci = s * zpt + t

            @pl.when(ci < nzc)
            def _():
                pltpu.sync_copy(acc.at[pl.ds(ci * zr, zr)],
                                part_hbm.at[c, pl.ds(ci * zr, zr)])

    return agg


_agg_full = _make_agg(128)
_agg_narrow = _make_agg(NCLS)


BN = 2000


def _prelu(v, a):
    return jnp.where(v >= 0, v, a * v)


def _combine_w(comp_ref, bases_ref):
    ws = []
    for r in range(NREL):
        W = comp_ref[r, 0] * bases_ref[0]
        for b in range(1, NREL):
            W = W + comp_ref[r, b] * bases_ref[b]
        ws.append(W)
    return ws


def _store_hr(hr_ref, r, hrr, slot):
    hr_ref[:, r, :] = hrr


def _mm1_body(num_ref, numm_ref, txt_ref, txtm_ref, x_ref,
              npw_ref, npb_ref, tpw_ref, tpb_ref, xpw_ref, xpb_ref, a_ref,
              comp_ref, bases_ref, root_ref, bias_ref, hr_ref,
              base_ref, *, slot):
    a = a_ref[...]
    nm = num_ref[...] * numm_ref[...]
    h0 = _prelu(nm * npw_ref[...] + npb_ref[...], a)
    t = jnp.dot(txt_ref[...] * txtm_ref[...], tpw_ref[...],
                preferred_element_type=_f32)
    t = _prelu(t + tpb_ref[...], a)
    xp = jnp.dot(x_ref[...], xpw_ref[...], preferred_element_type=_f32)
    h = h0 + t + xp + xpb_ref[...]
    ws = _combine_w(comp_ref, bases_ref)
    for r in range(NREL):
        hrr = jnp.dot(h, ws[r], preferred_element_type=_f32)
        _store_hr(hr_ref, r, hrr, slot)
    base_ref[...] = (jnp.dot(h, root_ref[...], preferred_element_type=_f32)
                     + bias_ref[...])


def _mmf_body(basep_ref, part_ref, act_ref, comp_ref, bases_ref, root_ref,
              bias_ref, hr_ref, base_ref, *, slot):
    p = part_ref[...]
    h = _prelu(basep_ref[...] + p[0] + p[1], act_ref[...])
    ws = _combine_w(comp_ref, bases_ref)
    for r in range(NREL):
        hrr = jnp.dot(h, ws[r], preferred_element_type=_f32)
        _store_hr(hr_ref, r, hrr, slot)
    base_ref[...] = (jnp.dot(h, root_ref[...], preferred_element_type=_f32)
                     + bias_ref[...])


def _hr_out(slot):
    d = NCLS if slot else HIDDEN
    return (pl.BlockSpec((BN, NREL, d), lambda i: (i, 0, 0)),
            jax.ShapeDtypeStruct((N, NREL, d), _f32))


def _mm1_call(num_x, num_mask, txt_x, txt_mask, x,
              npw, npb, tpw, tpb, xpw, xpb, a,
              comp, bases, root, bias, F, D, slot):
    grid = (N // BN,)
    blk = lambda shape: pl.BlockSpec(shape, lambda i: (i,) + (0,) * (len(shape) - 1))
    full = lambda shape: pl.BlockSpec(shape, lambda i: (0,) * len(shape))
    hr_spec, hr_shape = _hr_out(slot)
    return pl.pallas_call(
        functools.partial(_mm1_body, slot=slot),
        grid=grid,
        in_specs=[
            blk((BN, 1)), blk((BN, 1)), blk((BN, 384)), blk((BN, 1)),
            blk((BN, EMBED)),
            full((1, EMBED)), full((1, EMBED)), full((384, EMBED)),
            full((1, EMBED)), full((EMBED, HIDDEN)), full((1, HIDDEN)),
            full((1, EMBED)),
            pl.BlockSpec((NREL, NREL), lambda i: (0, 0),
                         memory_space=pltpu.SMEM),
            full((NREL, F, D)), full((F, D)), full((1, D)),
        ],
        out_specs=[hr_spec, blk((BN, D))],
        out_shape=[hr_shape, jax.ShapeDtypeStruct((N, D), _f32)],
    )(num_x, num_mask, txt_x, txt_mask, x, npw, npb, tpw, tpb, xpw, xpb, a,
      comp, bases, root, bias)


def _mmf_call(basep, part, act, comp, bases, root, bias, F, D, slot):
    grid = (N // BN,)
    blk = lambda shape: pl.BlockSpec(shape, lambda i: (i,) + (0,) * (len(shape) - 1))
    full = lambda shape: pl.BlockSpec(shape, lambda i: (0,) * len(shape))
    hr_spec, hr_shape = _hr_out(slot)
    return pl.pallas_call(
        functools.partial(_mmf_body, slot=slot),
        grid=grid,
        in_specs=[
            blk((BN, F)),
            pl.BlockSpec((NC, BN, 128), lambda i: (0, i, 0)),
            full((1, F)),
            pl.BlockSpec((NREL, NREL), lambda i: (0, 0),
                         memory_space=pltpu.SMEM),
            full((NREL, F, D)), full((F, D)), full((1, D)),
        ],
        out_specs=[hr_spec, blk((BN, D))],
        out_shape=[hr_shape, jax.ShapeDtypeStruct((N, D), _f32)],
    )(basep, part, act, comp, bases, root, bias)


def _final_body(base_ref, part_ref, out_ref):
    p = part_ref[...]
    v = base_ref[...] + p[0] + p[1]
    m = jnp.max(v, axis=1, keepdims=True)
    z = v - m
    lse = jnp.log(jnp.sum(jnp.exp(z), axis=1, keepdims=True))
    out_ref[...] = z - lse


def _final_call(base, part):
    return pl.pallas_call(
        _final_body,
        out_shape=jax.ShapeDtypeStruct((N, NCLS), _f32),
    )(base, part)


def kernel(x, num_x, num_mask, txt_x, txt_mask, edge_index, edge_type,
           num_proj_w, num_proj_b, txt_proj_w, txt_proj_b,
           node_proj_w, node_proj_b,
           input_act_a, act1_a, act2_a,
           comp1, bases1, root1, bias1,
           comp2, bases2, root2, bias2,
           comp3, bases3, root3, bias3):
    src = edge_index[0]
    dst = edge_index[1]
    et = edge_type

    w = _count_w(dst, et)

    hr1, b1 = _mm1_call(num_x, num_mask, txt_x, txt_mask.reshape(N, 1), x,
                        num_proj_w, num_proj_b.reshape(1, EMBED),
                        txt_proj_w, txt_proj_b.reshape(1, EMBED),
                        node_proj_w, node_proj_b.reshape(1, HIDDEN),
                        input_act_a.reshape(1, EMBED),
                        comp1, bases1, root1, bias1.reshape(1, HIDDEN),
                        EMBED, HIDDEN, False)
    p1 = _agg_full(hr1.reshape(N * NREL, HIDDEN), w, src, dst, et)

    hr2, b2 = _mmf_call(b1, p1, act1_a.reshape(1, HIDDEN), comp2, bases2,
                        root2, bias2.reshape(1, HIDDEN), HIDDEN, HIDDEN,
                        False)
    p2 = _agg_full(hr2.reshape(N * NREL, HIDDEN), w, src, dst, et)

    hr3, b3 = _mmf_call(b2, p2, act2_a.reshape(1, HIDDEN), comp3, bases3,
                        root3, bias3.reshape(1, NCLS), HIDDEN, NCLS, True)
    p3 = _agg_narrow(hr3.reshape(N * NREL, NCLS), w, src, dst, et)

    return _final_call(b3, p3)
